# baseline (speedup 1.0000x reference)
import jax
import jax.numpy as jnp
from jax import lax
from jax.experimental import pallas as pl
from jax.experimental.pallas import tpu as pltpu
from jax.experimental.pallas import tpu_sc as plsc

N_ATOMS = 100000
N_EDGES = 6400000
C = 8
CP = 16
NC = 2
NS = 16
NW = NC * NS
EB = 400
NB = 504
EPT = NB * EB
E_PAD = EPT * NW
N_PAD = 100096
RPT = N_PAD // NS
NSETS = 4


def _sc_body(charges, edata, recip, out,
             ed0, ed1, ed2, ed3, dd0, dd1, dd2, dd3, r0, r1, r2, r3, acc,
             e0, e1, e2, e3, g0, g1, g2, g3, s0, s1, s2, s3):
    ed = (ed0, ed1, ed2, ed3)
    dd = (dd0, dd1, dd2, dd3)
    rows = (r0, r1, r2, r3)
    esem = (e0, e1, e2, e3)
    gsem = (g0, g1, g2, g3)
    ssem = (s0, s1, s2, s3)

    cid = lax.axis_index("c")
    sid = lax.axis_index("s")

    def zrow(r, _):
        r0[r, :] = jnp.zeros((16,), jnp.float32)
        return 0
    lax.fori_loop(0, EB, zrow, 0)
    row0 = sid * RPT
    for z in range(15):
        pltpu.sync_copy(r0, acc.at[pl.ds(row0 + z * EB, EB)])
    pltpu.sync_copy(r0.at[pl.ds(0, 256)], acc.at[pl.ds(row0 + 6000, 256)])
    plsc.subcore_barrier()

    tile_base = (cid * NS + sid) * EPT

    for d in range(2):
        gr = 1 - d
        sr = d

        def stage(b, p):
            base = tile_base + b * EB
            pltpu.async_copy(edata.at[:, pl.ds(base, EB)], ed[p], esem[p])
            pltpu.async_copy(recip.at[pl.ds(base, EB)], dd[p], esem[p])

        def wait_stage(b, p):
            base = tile_base + b * EB
            pltpu.make_async_copy(
                edata.at[:, pl.ds(base, EB)], ed[p], esem[p]).wait()
            pltpu.make_async_copy(
                recip.at[pl.ds(base, EB)], dd[p], esem[p]).wait()

        def gather(p):
            pltpu.async_copy(charges.at[ed[p].at[gr]], rows[p], gsem[p])

        def wait_gather(p):
            pltpu.make_async_copy(
                charges.at[ed[p].at[gr]], rows[p], gsem[p]).wait()

        def scatter(p):
            pltpu.async_copy(rows[p], acc.at[ed[p].at[sr]], ssem[p], add=True)

        def wait_scatter(p):
            pltpu.make_async_copy(
                rows[p], acc.at[ed[p].at[sr]], ssem[p]).wait()

        def compute(p):
            rbuf = rows[p]
            ddp = dd[p]

            def scale(k, _):
                dvec = 1.0 / ddp[pl.ds(k * 16, 16)]
                for ei in range(16):
                    e = k * 16 + ei
                    rbuf[e, :] = rbuf[e, :] * dvec[ei]
                return 0
            lax.fori_loop(0, EB // 16, scale, 0)

        stage(0, 0)
        wait_stage(0, 0)
        gather(0)
        stage(1, 1)

        def g_body(g, _):
            for u in range(NSETS):
                b = 4 * g + u
                if u < 2:
                    @pl.when(g > 0)
                    def _():
                        wait_scatter((u + 2) % NSETS)
                else:
                    wait_scatter((u + 2) % NSETS)
                if u < 2:
                    stage(b + 2, (u + 2) % NSETS)
                else:
                    @pl.when(g < NB // 4 - 1)
                    def _():
                        stage(b + 2, (u + 2) % NSETS)
                if u < 3:
                    wait_stage(b + 1, (u + 1) % NSETS)
                    gather((u + 1) % NSETS)
                else:
                    @pl.when(g < NB // 4 - 1)
                    def _():
                        wait_stage(b + 1, 0)
                        gather(0)
                wait_gather(u)
                compute(u)
                scatter(u)
            return 0

        lax.fori_loop(0, NB // 4, g_body, 0)
        wait_scatter(2)
        wait_scatter(3)

    plsc.subcore_barrier()
    out_base = cid * N_PAD + row0
    pltpu.sync_copy(acc.at[pl.ds(row0, RPT)], out.at[pl.ds(out_base, RPT)])


def _combine(a_ref, b_ref, o_ref):
    o_ref[...] = (a_ref[...] + b_ref[...]) * 0.5


@jax.jit
def _impl(charges, neighbor_indices, neighbor_distances):
    ni = neighbor_indices.astype(jnp.int32)
    npad = E_PAD - N_EDGES
    spread = jnp.arange(npad, dtype=jnp.int32)
    ai = jnp.concatenate([ni[:, 0], N_ATOMS + spread % 96])
    aj = jnp.concatenate([ni[:, 1], N_ATOMS + (spread + 37) % 96])
    edata = jnp.stack([ai, aj], axis=0)
    recip = jnp.concatenate([neighbor_distances,
                             jnp.ones((npad,), jnp.float32)])
    charges_p = jnp.pad(charges, ((0, N_PAD - N_ATOMS), (0, CP - C)))

    mesh = plsc.VectorSubcoreMesh(core_axis_name="c", subcore_axis_name="s")
    sck = pl.kernel(
        _sc_body,
        out_type=jax.ShapeDtypeStruct((NC * N_PAD, CP), jnp.float32),
        mesh=mesh,
        scratch_types=(
            [pltpu.VMEM((2, EB), jnp.int32)] * NSETS
            + [pltpu.VMEM((EB,), jnp.float32)] * NSETS
            + [pltpu.VMEM((EB, CP), jnp.float32)] * NSETS
            + [pltpu.VMEM_SHARED((N_PAD, CP), jnp.float32)]
            + [pltpu.SemaphoreType.DMA] * 12
        ),
        compiler_params=pltpu.CompilerParams(use_tc_tiling_on_sc=False),
    )
    part = sck(charges_p, edata, recip)

    a = part[:N_PAD].reshape(12512, 128)
    b = part[N_PAD:].reshape(12512, 128)
    pot = pl.pallas_call(
        _combine,
        out_shape=jax.ShapeDtypeStruct((12512, 128), jnp.float32),
    )(a, b)
    return pot.reshape(N_PAD, CP)[:N_ATOMS, :C]


def kernel(charges, cell, positions, neighbor_indices, neighbor_distances):
    return _impl(charges, neighbor_indices, neighbor_distances)


# speedup vs baseline: 1.1270x; 1.1270x over previous
import jax
import jax.numpy as jnp
from jax import lax
from jax.experimental import pallas as pl
from jax.experimental.pallas import tpu as pltpu
from jax.experimental.pallas import tpu_sc as plsc

N_ATOMS = 100000
N_EDGES = 6400000
C = 8
CP = 16
NC = 2
NS = 16
NW = NC * NS
EB = 400
NB = 504
EPT = NB * EB
E_PAD = EPT * NW
N_PAD = 100096
RPT = N_PAD // NS
NSETS = 4


def _sc_body(charges, ai, aj, recip, out,
             ed0, ed1, ed2, ed3, dd0, dd1, dd2, dd3, r0, r1, r2, r3, acc,
             e0, e1, e2, e3, g0, g1, g2, g3, s0, s1, s2, s3):
    ed = (ed0, ed1, ed2, ed3)
    dd = (dd0, dd1, dd2, dd3)
    rows = (r0, r1, r2, r3)
    esem = (e0, e1, e2, e3)
    gsem = (g0, g1, g2, g3)
    ssem = (s0, s1, s2, s3)

    cid = lax.axis_index("c")
    sid = lax.axis_index("s")

    def zrow(r, _):
        r0[r, :] = jnp.zeros((16,), jnp.float32)
        return 0
    lax.fori_loop(0, EB, zrow, 0)
    row0 = sid * RPT
    for z in range(15):
        pltpu.sync_copy(r0, acc.at[pl.ds(row0 + z * EB, EB)])
    pltpu.sync_copy(r0.at[pl.ds(0, 256)], acc.at[pl.ds(row0 + 6000, 256)])
    plsc.subcore_barrier()

    tile_base = (cid * NS + sid) * EPT

    for d in range(2):
        gr = 1 - d
        sr = d

        def stage(b, p):
            base = tile_base + b * EB
            pltpu.async_copy(ai.at[pl.ds(base, EB)], ed[p].at[0], esem[p])
            pltpu.async_copy(aj.at[pl.ds(base, EB)], ed[p].at[1], esem[p])
            pltpu.async_copy(recip.at[pl.ds(base, EB)], dd[p], esem[p])

        def wait_stage(b, p):
            base = tile_base + b * EB
            pltpu.make_async_copy(
                ai.at[pl.ds(base, EB)], ed[p].at[0], esem[p]).wait()
            pltpu.make_async_copy(
                aj.at[pl.ds(base, EB)], ed[p].at[1], esem[p]).wait()
            pltpu.make_async_copy(
                recip.at[pl.ds(base, EB)], dd[p], esem[p]).wait()

        def gather(p):
            pltpu.async_copy(charges.at[ed[p].at[gr]], rows[p], gsem[p])

        def wait_gather(p):
            pltpu.make_async_copy(
                charges.at[ed[p].at[gr]], rows[p], gsem[p]).wait()

        def scatter(p):
            pltpu.async_copy(rows[p], acc.at[ed[p].at[sr]], ssem[p], add=True)

        def wait_scatter(p):
            pltpu.make_async_copy(
                rows[p], acc.at[ed[p].at[sr]], ssem[p]).wait()

        def compute(p):
            rbuf = rows[p]
            ddp = dd[p]

            def scale(k, _):
                dvec = ddp[pl.ds(k * 16, 16)]
                for ei in range(16):
                    e = k * 16 + ei
                    rbuf[e, :] = rbuf[e, :] * dvec[ei]
                return 0
            lax.fori_loop(0, EB // 16, scale, 0)

        stage(0, 0)
        wait_stage(0, 0)
        gather(0)
        stage(1, 1)

        def g_body(g, _):
            for u in range(NSETS):
                b = 4 * g + u
                if u < 2:
                    @pl.when(g > 0)
                    def _():
                        wait_scatter((u + 2) % NSETS)
                else:
                    wait_scatter((u + 2) % NSETS)
                if u < 2:
                    stage(b + 2, (u + 2) % NSETS)
                else:
                    @pl.when(g < NB // 4 - 1)
                    def _():
                        stage(b + 2, (u + 2) % NSETS)
                if u < 3:
                    wait_stage(b + 1, (u + 1) % NSETS)
                    gather((u + 1) % NSETS)
                else:
                    @pl.when(g < NB // 4 - 1)
                    def _():
                        wait_stage(b + 1, 0)
                        gather(0)
                wait_gather(u)
                compute(u)
                scatter(u)
            return 0

        lax.fori_loop(0, NB // 4, g_body, 0)
        wait_scatter(2)
        wait_scatter(3)

    plsc.subcore_barrier()
    out_base = cid * N_PAD + row0
    pltpu.sync_copy(acc.at[pl.ds(row0, RPT)], out.at[pl.ds(out_base, RPT)])


def _recip(d_ref, o_ref):
    o_ref[...] = 1.0 / d_ref[...]


def _combine(a_ref, b_ref, o_ref):
    o_ref[...] = (a_ref[:, :C] + b_ref[:, :C]) * 0.5


@jax.jit
def _impl(charges, neighbor_indices, neighbor_distances):
    ni = neighbor_indices.astype(jnp.int32)
    npad = E_PAD - N_EDGES
    spread = jnp.arange(npad, dtype=jnp.int32)
    ai = jnp.concatenate([ni[:, 0], N_ATOMS + spread % 96])
    aj = jnp.concatenate([ni[:, 1], N_ATOMS + (spread + 37) % 96])
    dist = jnp.concatenate([neighbor_distances,
                            jnp.ones((npad,), jnp.float32)])
    recip = pl.pallas_call(
        _recip,
        grid=(10,),
        in_specs=[pl.BlockSpec((5040, 128), lambda i: (i, 0))],
        out_specs=pl.BlockSpec((5040, 128), lambda i: (i, 0)),
        out_shape=jax.ShapeDtypeStruct((50400, 128), jnp.float32),
    )(dist.reshape(50400, 128)).reshape(E_PAD)
    charges_p = jnp.pad(charges, ((0, N_PAD - N_ATOMS), (0, CP - C)))

    mesh = plsc.VectorSubcoreMesh(core_axis_name="c", subcore_axis_name="s")
    sck = pl.kernel(
        _sc_body,
        out_type=jax.ShapeDtypeStruct((NC * N_PAD, CP), jnp.float32),
        mesh=mesh,
        scratch_types=(
            [pltpu.VMEM((2, EB), jnp.int32)] * NSETS
            + [pltpu.VMEM((EB,), jnp.float32)] * NSETS
            + [pltpu.VMEM((EB, CP), jnp.float32)] * NSETS
            + [pltpu.VMEM_SHARED((N_PAD, CP), jnp.float32)]
            + [pltpu.SemaphoreType.DMA] * 12
        ),
        compiler_params=pltpu.CompilerParams(use_tc_tiling_on_sc=False),
    )
    part = sck(charges_p, ai, aj, recip)

    pot = pl.pallas_call(
        _combine,
        grid=(100,),
        in_specs=[pl.BlockSpec((1000, CP), lambda i: (i, 0)),
                  pl.BlockSpec((1000, CP), lambda i: (i, 0))],
        out_specs=pl.BlockSpec((1000, C), lambda i: (i, 0)),
        out_shape=jax.ShapeDtypeStruct((N_ATOMS, C), jnp.float32),
    )(part[:N_PAD], part[N_PAD:])
    return pot


def kernel(charges, cell, positions, neighbor_indices, neighbor_distances):
    return _impl(charges, neighbor_indices, neighbor_distances)


# speedup vs baseline: 1.1276x; 1.0005x over previous
import jax
import jax.numpy as jnp
from jax import lax
from jax.experimental import pallas as pl
from jax.experimental.pallas import tpu as pltpu
from jax.experimental.pallas import tpu_sc as plsc

N_ATOMS = 100000
N_EDGES = 6400000
C = 8
CP = 16
NC = 2
NS = 16
NW = NC * NS
EB = 400
NB = 504
EPT = NB * EB
E_PAD = EPT * NW
N_PAD = 100096
RPT = N_PAD // NS
NSETS = 4


def _sc_body(charges, ai, aj, recip, out,
             ed0, ed1, ed2, ed3, dd0, dd1, dd2, dd3, r0, r1, r2, r3, acc,
             e0, e1, e2, e3, g0, g1, g2, g3, s0, s1, s2, s3):
    ed = (ed0, ed1, ed2, ed3)
    dd = (dd0, dd1, dd2, dd3)
    rows = (r0, r1, r2, r3)
    esem = (e0, e1, e2, e3)
    gsem = (g0, g1, g2, g3)
    ssem = (s0, s1, s2, s3)

    cid = lax.axis_index("c")
    sid = lax.axis_index("s")

    def zrow(r, _):
        r0[r, :] = jnp.zeros((16,), jnp.float32)
        return 0
    lax.fori_loop(0, EB, zrow, 0)
    row0 = sid * RPT
    for z in range(15):
        pltpu.sync_copy(r0, acc.at[pl.ds(row0 + z * EB, EB)])
    pltpu.sync_copy(r0.at[pl.ds(0, 256)], acc.at[pl.ds(row0 + 6000, 256)])
    plsc.subcore_barrier()

    tile_base = (cid * NS + sid) * EPT

    for d in range(2):
        gr = 1 - d
        sr = d

        def stage(b, p):
            base = tile_base + b * EB
            pltpu.async_copy(ai.at[pl.ds(base, EB)], ed[p].at[0], esem[p])
            pltpu.async_copy(aj.at[pl.ds(base, EB)], ed[p].at[1], esem[p])
            pltpu.async_copy(recip.at[pl.ds(base, EB)], dd[p], esem[p])

        def wait_stage(b, p):
            base = tile_base + b * EB
            pltpu.make_async_copy(
                ai.at[pl.ds(base, EB)], ed[p].at[0], esem[p]).wait()
            pltpu.make_async_copy(
                aj.at[pl.ds(base, EB)], ed[p].at[1], esem[p]).wait()
            pltpu.make_async_copy(
                recip.at[pl.ds(base, EB)], dd[p], esem[p]).wait()

        def gather(p):
            pltpu.async_copy(charges.at[ed[p].at[gr]], rows[p], gsem[p])

        def wait_gather(p):
            pltpu.make_async_copy(
                charges.at[ed[p].at[gr]], rows[p], gsem[p]).wait()

        def scatter(p):
            pltpu.async_copy(rows[p], acc.at[ed[p].at[sr]], ssem[p], add=True)

        def wait_scatter(p):
            pltpu.make_async_copy(
                rows[p], acc.at[ed[p].at[sr]], ssem[p]).wait()

        def compute(p):
            rbuf = rows[p]
            ddp = dd[p]

            def scale(k, _):
                dvec = ddp[pl.ds(k * 16, 16)]
                for ei in range(16):
                    e = k * 16 + ei
                    sv = lax.gather(
                        dvec, jnp.full((16, 1), ei, jnp.int32),
                        lax.GatherDimensionNumbers(offset_dims=(),
                                                   collapsed_slice_dims=(0,),
                                                   start_index_map=(0,)),
                        (1,), mode=lax.GatherScatterMode.PROMISE_IN_BOUNDS)
                    rbuf[e, :] = rbuf[e, :] * sv
                return 0
            lax.fori_loop(0, EB // 16, scale, 0)

        stage(0, 0)
        wait_stage(0, 0)
        gather(0)
        stage(1, 1)

        def g_body(g, _):
            for u in range(NSETS):
                b = 4 * g + u
                if u < 2:
                    @pl.when(g > 0)
                    def _():
                        wait_scatter((u + 2) % NSETS)
                else:
                    wait_scatter((u + 2) % NSETS)
                if u < 2:
                    stage(b + 2, (u + 2) % NSETS)
                else:
                    @pl.when(g < NB // 4 - 1)
                    def _():
                        stage(b + 2, (u + 2) % NSETS)
                if u < 3:
                    wait_stage(b + 1, (u + 1) % NSETS)
                    gather((u + 1) % NSETS)
                else:
                    @pl.when(g < NB // 4 - 1)
                    def _():
                        wait_stage(b + 1, 0)
                        gather(0)
                wait_gather(u)
                compute(u)
                scatter(u)
            return 0

        lax.fori_loop(0, NB // 4, g_body, 0)
        wait_scatter(2)
        wait_scatter(3)

    plsc.subcore_barrier()
    out_base = cid * N_PAD + row0
    pltpu.sync_copy(acc.at[pl.ds(row0, RPT)], out.at[pl.ds(out_base, RPT)])


def _recip(d_ref, o_ref):
    o_ref[...] = 1.0 / d_ref[...]


def _combine(a_ref, b_ref, o_ref):
    o_ref[...] = (a_ref[:, :C] + b_ref[:, :C]) * 0.5


@jax.jit
def _impl(charges, neighbor_indices, neighbor_distances):
    ni = neighbor_indices.astype(jnp.int32)
    npad = E_PAD - N_EDGES
    spread = jnp.arange(npad, dtype=jnp.int32)
    ai = jnp.concatenate([ni[:, 0], N_ATOMS + spread % 96])
    aj = jnp.concatenate([ni[:, 1], N_ATOMS + (spread + 37) % 96])
    dist = jnp.concatenate([neighbor_distances,
                            jnp.ones((npad,), jnp.float32)])
    recip = pl.pallas_call(
        _recip,
        grid=(10,),
        in_specs=[pl.BlockSpec((5040, 128), lambda i: (i, 0))],
        out_specs=pl.BlockSpec((5040, 128), lambda i: (i, 0)),
        out_shape=jax.ShapeDtypeStruct((50400, 128), jnp.float32),
    )(dist.reshape(50400, 128)).reshape(E_PAD)
    charges_p = jnp.pad(charges, ((0, N_PAD - N_ATOMS), (0, CP - C)))

    mesh = plsc.VectorSubcoreMesh(core_axis_name="c", subcore_axis_name="s")
    sck = pl.kernel(
        _sc_body,
        out_type=jax.ShapeDtypeStruct((NC * N_PAD, CP), jnp.float32),
        mesh=mesh,
        scratch_types=(
            [pltpu.VMEM((2, EB), jnp.int32)] * NSETS
            + [pltpu.VMEM((EB,), jnp.float32)] * NSETS
            + [pltpu.VMEM((EB, CP), jnp.float32)] * NSETS
            + [pltpu.VMEM_SHARED((N_PAD, CP), jnp.float32)]
            + [pltpu.SemaphoreType.DMA] * 12
        ),
        compiler_params=pltpu.CompilerParams(use_tc_tiling_on_sc=False),
    )
    part = sck(charges_p, ai, aj, recip)

    pot = pl.pallas_call(
        _combine,
        grid=(100,),
        in_specs=[pl.BlockSpec((1000, CP), lambda i: (i, 0)),
                  pl.BlockSpec((1000, CP), lambda i: (i, 0))],
        out_specs=pl.BlockSpec((1000, C), lambda i: (i, 0)),
        out_shape=jax.ShapeDtypeStruct((N_ATOMS, C), jnp.float32),
    )(part[:N_PAD], part[N_PAD:])
    return pot


def kernel(charges, cell, positions, neighbor_indices, neighbor_distances):
    return _impl(charges, neighbor_indices, neighbor_distances)
